# baseline (speedup 1.0000x reference)
import functools

import jax
import jax.numpy as jnp
from jax import lax
from jax.experimental import pallas as pl
from jax.experimental.pallas import tpu as pltpu
from jax.experimental.pallas import tpu_sc as plsc

N = 10000
E = 160000
H = 256
G = 64
HALF = 128
NC = 2
NS = 16
EPT = E // NS
CH = 80
NCH = EPT // CH
CHP = 128
EPTP = 10240
NCHP = EPTP // CHP
NPAD = N + 16
RPT = 624
TAIL = N - NS * RPT
NBUF = 2
BN = 1000
NB = N // BN

def _sc_degree_body(dst_hbm, ones_hbm, zeros_hbm, deg_hbm, dst_v, ones_v, deg_sh):
    c = lax.axis_index("c")
    s = lax.axis_index("s")
    pltpu.sync_copy(zeros_hbm, deg_sh.at[pl.ds(s * RPT, RPT)])

    @pl.when(s == 0)
    def _():
        pltpu.sync_copy(
            zeros_hbm.at[pl.ds(0, TAIL)], deg_sh.at[pl.ds(NS * RPT, TAIL)]
        )

    pltpu.sync_copy(dst_hbm.at[s], dst_v)
    pltpu.sync_copy(ones_hbm, ones_v)
    plsc.subcore_barrier()

    @pl.loop(0, NCH // 2)
    def _(j):
        pltpu.sync_copy(ones_v, deg_sh.at[dst_v.at[2 * j + c]], add=True)

    @pl.when(c == 0)
    def _():
        pltpu.sync_copy(ones_v, deg_sh.at[dst_v.at[NCH - 1]], add=True)

    plsc.subcore_barrier()
    pltpu.sync_copy(
        deg_sh.at[pl.ds(s * RPT, RPT)],
        deg_hbm.at[pl.ds(c * N + s * RPT, RPT)],
    )

    @pl.when(s == 0)
    def _():
        pltpu.sync_copy(
            deg_sh.at[pl.ds(NS * RPT, TAIL)],
            deg_hbm.at[pl.ds(c * N + NS * RPT, TAIL)],
        )


def _sc_agg_body(hflat_hbm, src2_hbm, dstf_hbm, zeros_hbm, agg_hbm,
                 src_v, dst_0, dst_1, rows_0, rows_1, acc_sh,
                 gsem_0, gsem_1, isem_0, isem_1):
    c = lax.axis_index("c")
    s = lax.axis_index("s")
    dsts = (dst_0, dst_1)
    rows = (rows_0, rows_1)
    gsems = (gsem_0, gsem_1)
    isems = (isem_0, isem_1)
    ebase = s * EPTP
    pltpu.sync_copy(zeros_hbm, acc_sh.at[pl.ds(s * RPT, RPT)])

    @pl.when(s == 0)
    def _():
        pltpu.sync_copy(
            zeros_hbm.at[pl.ds(0, NPAD - NS * RPT)],
            acc_sh.at[pl.ds(NS * RPT, NPAD - NS * RPT)],
        )

    pltpu.sync_copy(src2_hbm.at[c * NS + s], src_v)
    plsc.subcore_barrier()

    for b in range(NBUF):
        pltpu.async_copy(
            dstf_hbm.at[pl.ds(ebase + b * CHP, CHP)], dsts[b], isems[b]
        )
        pltpu.async_copy(
            hflat_hbm.at[src_v.at[pl.ds(b * CHP, CHP)]], rows[b], gsems[b]
        )

    @pl.loop(0, NCHP // NBUF)
    def _(j):
        for b in range(NBUF):
            k = NBUF * j + b
            pltpu.make_async_copy(
                dstf_hbm.at[pl.ds(ebase + k * CHP, CHP)], dsts[b], isems[b]
            ).wait()
            pltpu.make_async_copy(
                hflat_hbm.at[src_v.at[pl.ds(k * CHP, CHP)]], rows[b], gsems[b]
            ).wait()
            pltpu.sync_copy(rows[b], acc_sh.at[dsts[b]], add=True)

            @pl.when(k + NBUF < NCHP)
            def _(b=b, k=k):
                pltpu.async_copy(
                    dstf_hbm.at[pl.ds(ebase + (k + NBUF) * CHP, CHP)],
                    dsts[b], isems[b],
                )
                pltpu.async_copy(
                    hflat_hbm.at[src_v.at[pl.ds((k + NBUF) * CHP, CHP)]],
                    rows[b], gsems[b],
                )

    plsc.subcore_barrier()
    pltpu.sync_copy(
        acc_sh.at[pl.ds(s * RPT, RPT)],
        agg_hbm.at[pl.ds(c * N + s * RPT, RPT)],
    )

    @pl.when(s == 0)
    def _():
        pltpu.sync_copy(
            acc_sh.at[pl.ds(NS * RPT, TAIL)],
            agg_hbm.at[pl.ds(c * N + NS * RPT, TAIL)],
        )


@functools.cache
def _build_sc_kernels():
    mesh = plsc.VectorSubcoreMesh(
        core_axis_name="c", subcore_axis_name="s", num_cores=NC, num_subcores=NS
    )
    sc_degree = pl.kernel(
        _sc_degree_body,
        out_type=jax.ShapeDtypeStruct((NC * N, HALF), jnp.float32),
        mesh=mesh,
        scratch_types=[
            pltpu.VMEM((NCH, CH), jnp.int32),
            pltpu.VMEM((CH, HALF), jnp.float32),
            pltpu.VMEM_SHARED((N, HALF), jnp.float32),
        ],
    )
    sc_agg = pl.kernel(
        _sc_agg_body,
        out_type=jax.ShapeDtypeStruct((NC * N, HALF), jnp.float32),
        mesh=mesh,
        scratch_types=(
            [pltpu.VMEM((EPTP,), jnp.int32)]
            + [pltpu.VMEM((CHP,), jnp.int32)] * NBUF
            + [pltpu.VMEM((CHP, HALF), jnp.float32)] * NBUF
            + [pltpu.VMEM_SHARED((NPAD, HALF), jnp.float32)]
            + [pltpu.SemaphoreType.DMA] * (2 * NBUF)
        ),
    )
    return sc_degree, sc_agg


def _dense_block(a0, a1, dg0, dg1, h0, h1, Wl, bl, Wr):
    deg = jnp.maximum(dg0[0][:, 0:1] + dg1[0][:, 0:1], 1.0)
    inv = 1.0 / deg
    return (
        jnp.dot(a0[...] * inv, Wl[0:HALF, :], preferred_element_type=jnp.float32)
        + jnp.dot(a1[...] * inv, Wl[HALF:H, :], preferred_element_type=jnp.float32)
        + jnp.dot(h0[...], Wr[0:HALF, :], preferred_element_type=jnp.float32)
        + jnp.dot(h1[...], Wr[HALF:H, :], preferred_element_type=jnp.float32)
        + bl[...]
    )


def _mm_relu_body(a0, a1, dg0, dg1, h0, h1, Wl, bl, Wr, out):
    acc = jnp.maximum(_dense_block(a0, a1, dg0, dg1, h0, h1, Wl, bl, Wr), 0.0)
    out[0] = acc[:, 0:HALF]
    out[1] = acc[:, HALF:H]


def _mm_pool_head_body(a0, a1, dg0, dg1, h0, h1, Wl, bl, Wr, batch, hW, hb,
                       out, pooled_acc, cnt_acc):
    b = pl.program_id(0)
    acc = _dense_block(a0, a1, dg0, dg1, h0, h1, Wl, bl, Wr)
    onehot = (
        batch[...] == lax.broadcasted_iota(jnp.int32, (BN, G), 1)
    ).astype(jnp.float32)

    @pl.when(b == 0)
    def _():
        pooled_acc[...] = jnp.zeros_like(pooled_acc)
        cnt_acc[...] = jnp.zeros_like(cnt_acc)

    pooled_acc[...] += lax.dot_general(
        onehot, acc, (((0,), (0,)), ((), ())),
        preferred_element_type=jnp.float32,
    )
    cnt_acc[...] += lax.dot_general(
        onehot, jnp.ones((BN, HALF), jnp.float32), (((0,), (0,)), ((), ())),
        preferred_element_type=jnp.float32,
    )

    @pl.when(b == NB - 1)
    def _():
        cnt = jnp.maximum(cnt_acc[:, 0:1], 1.0)
        pooled = pooled_acc[...] / cnt
        out[...] = (
            jnp.dot(pooled, hW[...], preferred_element_type=jnp.float32) + hb[...]
        )


def _row_spec(off):
    return pl.BlockSpec((BN, HALF), lambda b, off=off: (b + off, 0))


_common_in_specs = [
    _row_spec(0),
    _row_spec(NB),
    pl.BlockSpec((1, BN, HALF), lambda b: (0, b, 0)),
    pl.BlockSpec((1, BN, HALF), lambda b: (1, b, 0)),
    _row_spec(0),
    _row_spec(NB),
    pl.BlockSpec((H, H), lambda b: (0, 0)),
    pl.BlockSpec((1, H), lambda b: (0, 0)),
    pl.BlockSpec((H, H), lambda b: (0, 0)),
]

_mm_relu = pl.pallas_call(
    _mm_relu_body,
    grid=(NB,),
    in_specs=_common_in_specs,
    out_specs=pl.BlockSpec((2, BN, HALF), lambda b: (0, b, 0)),
    out_shape=jax.ShapeDtypeStruct((2, N, HALF), jnp.float32),
)

_mm_pool_head = pl.pallas_call(
    _mm_pool_head_body,
    grid=(NB,),
    in_specs=_common_in_specs + [
        pl.BlockSpec((BN, 1), lambda b: (b, 0)),
        pl.BlockSpec((H, HALF), lambda b: (0, 0)),
        pl.BlockSpec((1, HALF), lambda b: (0, 0)),
    ],
    out_specs=pl.BlockSpec((G, HALF), lambda b: (0, 0)),
    out_shape=jax.ShapeDtypeStruct((G, HALF), jnp.float32),
    scratch_shapes=[
        pltpu.VMEM((G, H), jnp.float32),
        pltpu.VMEM((G, HALF), jnp.float32),
    ],
)


def kernel(x, edge_index, batch, Wl0, bl0, Wr0, Wl1, bl1, Wr1, Wl2, bl2, Wr2,
           Wl3, bl3, Wr3, head_W, head_b):
    _sc_degree, _sc_agg = _build_sc_kernels()
    src = edge_index[0].astype(jnp.int32)
    dst = edge_index[1].astype(jnp.int32)
    src_pad = jnp.concatenate(
        [src.reshape(NS, EPT), jnp.zeros((NS, EPTP - EPT), jnp.int32)], axis=1)
    src2 = jnp.concatenate([src_pad, src_pad + N], axis=0)
    dst_flat = jnp.concatenate(
        [dst.reshape(NS, EPT), jnp.full((NS, EPTP - EPT), N, jnp.int32)],
        axis=1).reshape(NS * EPTP)
    dst_rs = dst.reshape(NS, NCH, CH)

    ones_rows = jnp.ones((CH, HALF), jnp.float32)
    zeros_half = jnp.zeros((RPT, HALF), jnp.float32)

    deg2 = _sc_degree(dst_rs, ones_rows, zeros_half).reshape(NC, N, HALF)

    hflat = jnp.concatenate([x[:, 0:HALF], x[:, HALF:H]], axis=0)

    batch2d = batch.astype(jnp.int32).reshape(N, 1)
    hW_pad = jnp.pad(head_W, ((0, 0), (0, HALF - 1)))
    hb_pad = jnp.pad(head_b.reshape(1, 1), ((0, 0), (0, HALF - 1)))

    layers = ((Wl0, bl0, Wr0), (Wl1, bl1, Wr1), (Wl2, bl2, Wr2))
    for Wl, bl, Wr in layers:
        agg = _sc_agg(hflat, src2, dst_flat, zeros_half)
        hflat = _mm_relu(agg, agg, deg2, deg2, hflat, hflat,
                         Wl, bl.reshape(1, H), Wr).reshape(NC * N, HALF)

    agg = _sc_agg(hflat, src2, dst_flat, zeros_half)
    out = _mm_pool_head(agg, agg, deg2, deg2, hflat, hflat,
                        Wl3, bl3.reshape(1, H), Wr3, batch2d, hW_pad, hb_pad)
    return out[:, 0:1]


# speedup vs baseline: 1.5716x; 1.5716x over previous
import functools

import jax
import jax.numpy as jnp
from jax import lax
from jax.experimental import pallas as pl
from jax.experimental.pallas import tpu as pltpu
from jax.experimental.pallas import tpu_sc as plsc

N = 10000
E = 160000
H = 256
G = 64
HALF = 128
NC = 2
NS = 16
EPT = E // NS
CH = 80
NCH = EPT // CH
CHP = 80
EPTP = 10080
NCHP = EPTP // CHP
NPAD = N + 16
RPT = 624
TAIL = N - NS * RPT
NRING = 3
BN = 1000
NB = N // BN

def _sc_degree_body(dst_hbm, ones_hbm, zeros_hbm, deg_hbm, dst_v, ones_v, deg_sh):
    c = lax.axis_index("c")
    s = lax.axis_index("s")
    pltpu.sync_copy(zeros_hbm, deg_sh.at[pl.ds(s * RPT, RPT)])

    @pl.when(s == 0)
    def _():
        pltpu.sync_copy(
            zeros_hbm.at[pl.ds(0, TAIL)], deg_sh.at[pl.ds(NS * RPT, TAIL)]
        )

    pltpu.sync_copy(dst_hbm.at[s], dst_v)
    pltpu.sync_copy(ones_hbm, ones_v)
    plsc.subcore_barrier()

    @pl.loop(0, NCH // 2)
    def _(j):
        pltpu.sync_copy(ones_v, deg_sh.at[dst_v.at[2 * j + c]], add=True)

    @pl.when(c == 0)
    def _():
        pltpu.sync_copy(ones_v, deg_sh.at[dst_v.at[NCH - 1]], add=True)

    plsc.subcore_barrier()
    pltpu.sync_copy(
        deg_sh.at[pl.ds(s * RPT, RPT)],
        deg_hbm.at[pl.ds(c * N + s * RPT, RPT)],
    )

    @pl.when(s == 0)
    def _():
        pltpu.sync_copy(
            deg_sh.at[pl.ds(NS * RPT, TAIL)],
            deg_hbm.at[pl.ds(c * N + NS * RPT, TAIL)],
        )


def _sc_agg_body(hflat_hbm, src2_hbm, dstf_hbm, zeros_hbm, agg_hbm,
                 src_v, dst_0, dst_1, dst_2, rows_0, rows_1, rows_2, acc_sh,
                 gsem_0, gsem_1, gsem_2, isem_0, isem_1, isem_2,
                 ssem_0, ssem_1, ssem_2):
    c = lax.axis_index("c")
    s = lax.axis_index("s")
    dsts = (dst_0, dst_1, dst_2)
    rows = (rows_0, rows_1, rows_2)
    gsems = (gsem_0, gsem_1, gsem_2)
    isems = (isem_0, isem_1, isem_2)
    ssems = (ssem_0, ssem_1, ssem_2)
    ebase = s * EPTP
    pltpu.sync_copy(zeros_hbm, acc_sh.at[pl.ds(s * RPT, RPT)])

    @pl.when(s == 0)
    def _():
        pltpu.sync_copy(
            zeros_hbm.at[pl.ds(0, NPAD - NS * RPT)],
            acc_sh.at[pl.ds(NS * RPT, NPAD - NS * RPT)],
        )

    pltpu.sync_copy(src2_hbm.at[c * NS + s], src_v)
    plsc.subcore_barrier()

    def issue(slot, k):
        pltpu.async_copy(
            dstf_hbm.at[pl.ds(ebase + k * CHP, CHP)], dsts[slot], isems[slot]
        )
        pltpu.async_copy(
            hflat_hbm.at[src_v.at[pl.ds(k * CHP, CHP)]], rows[slot], gsems[slot]
        )

    issue(0, 0)
    issue(1, 1)

    @pl.loop(0, NCHP // NRING)
    def _(j):
        for b in range(NRING):
            k = NRING * j + b
            nslot = (b + 2) % NRING

            @pl.when(k + 2 < NCHP)
            def _(b=b, k=k, nslot=nslot):
                @pl.when(k >= 1)
                def _():
                    pltpu.make_async_copy(
                        rows[nslot], acc_sh.at[dsts[nslot]], ssems[nslot]
                    ).wait()

                issue(nslot, k + 2)

            pltpu.make_async_copy(
                dstf_hbm.at[pl.ds(ebase + k * CHP, CHP)], dsts[b], isems[b]
            ).wait()
            pltpu.make_async_copy(
                hflat_hbm.at[src_v.at[pl.ds(k * CHP, CHP)]], rows[b], gsems[b]
            ).wait()
            pltpu.async_copy(rows[b], acc_sh.at[dsts[b]], ssems[b], add=True)

    for b in range(NRING):
        pltpu.make_async_copy(rows[b], acc_sh.at[dsts[b]], ssems[b]).wait()

    plsc.subcore_barrier()
    pltpu.sync_copy(
        acc_sh.at[pl.ds(s * RPT, RPT)],
        agg_hbm.at[pl.ds(c * N + s * RPT, RPT)],
    )

    @pl.when(s == 0)
    def _():
        pltpu.sync_copy(
            acc_sh.at[pl.ds(NS * RPT, TAIL)],
            agg_hbm.at[pl.ds(c * N + NS * RPT, TAIL)],
        )


@functools.cache
def _build_sc_kernels():
    mesh = plsc.VectorSubcoreMesh(
        core_axis_name="c", subcore_axis_name="s", num_cores=NC, num_subcores=NS
    )
    sc_degree = pl.kernel(
        _sc_degree_body,
        out_type=jax.ShapeDtypeStruct((NC * N, HALF), jnp.float32),
        mesh=mesh,
        scratch_types=[
            pltpu.VMEM((NCH, CH), jnp.int32),
            pltpu.VMEM((CH, HALF), jnp.float32),
            pltpu.VMEM_SHARED((N, HALF), jnp.float32),
        ],
    )
    sc_agg = pl.kernel(
        _sc_agg_body,
        out_type=jax.ShapeDtypeStruct((NC * N, HALF), jnp.float32),
        mesh=mesh,
        scratch_types=(
            [pltpu.VMEM((EPTP,), jnp.int32)]
            + [pltpu.VMEM((CHP,), jnp.int32)] * NRING
            + [pltpu.VMEM((CHP, HALF), jnp.float32)] * NRING
            + [pltpu.VMEM_SHARED((NPAD, HALF), jnp.float32)]
            + [pltpu.SemaphoreType.DMA] * (3 * NRING)
        ),
    )
    return sc_degree, sc_agg


def _dense_block(a0, a1, dg0, dg1, h0, h1, Wl, bl, Wr):
    deg = jnp.maximum(dg0[0][:, 0:1] + dg1[0][:, 0:1], 1.0)
    inv = 1.0 / deg
    return (
        jnp.dot(a0[...] * inv, Wl[0:HALF, :], preferred_element_type=jnp.float32)
        + jnp.dot(a1[...] * inv, Wl[HALF:H, :], preferred_element_type=jnp.float32)
        + jnp.dot(h0[...], Wr[0:HALF, :], preferred_element_type=jnp.float32)
        + jnp.dot(h1[...], Wr[HALF:H, :], preferred_element_type=jnp.float32)
        + bl[...]
    )


def _mm_relu_body(a0, a1, dg0, dg1, h0, h1, Wl, bl, Wr, out):
    acc = jnp.maximum(_dense_block(a0, a1, dg0, dg1, h0, h1, Wl, bl, Wr), 0.0)
    out[0] = acc[:, 0:HALF]
    out[1] = acc[:, HALF:H]


def _mm_pool_head_body(a0, a1, dg0, dg1, h0, h1, Wl, bl, Wr, batch, hW, hb,
                       out, pooled_acc, cnt_acc):
    b = pl.program_id(0)
    acc = _dense_block(a0, a1, dg0, dg1, h0, h1, Wl, bl, Wr)
    onehot = (
        batch[...] == lax.broadcasted_iota(jnp.int32, (BN, G), 1)
    ).astype(jnp.float32)

    @pl.when(b == 0)
    def _():
        pooled_acc[...] = jnp.zeros_like(pooled_acc)
        cnt_acc[...] = jnp.zeros_like(cnt_acc)

    pooled_acc[...] += lax.dot_general(
        onehot, acc, (((0,), (0,)), ((), ())),
        preferred_element_type=jnp.float32,
    )
    cnt_acc[...] += lax.dot_general(
        onehot, jnp.ones((BN, HALF), jnp.float32), (((0,), (0,)), ((), ())),
        preferred_element_type=jnp.float32,
    )

    @pl.when(b == NB - 1)
    def _():
        cnt = jnp.maximum(cnt_acc[:, 0:1], 1.0)
        pooled = pooled_acc[...] / cnt
        out[...] = (
            jnp.dot(pooled, hW[...], preferred_element_type=jnp.float32) + hb[...]
        )


def _row_spec(off):
    return pl.BlockSpec((BN, HALF), lambda b, off=off: (b + off, 0))


_common_in_specs = [
    _row_spec(0),
    _row_spec(NB),
    pl.BlockSpec((1, BN, HALF), lambda b: (0, b, 0)),
    pl.BlockSpec((1, BN, HALF), lambda b: (1, b, 0)),
    _row_spec(0),
    _row_spec(NB),
    pl.BlockSpec((H, H), lambda b: (0, 0)),
    pl.BlockSpec((1, H), lambda b: (0, 0)),
    pl.BlockSpec((H, H), lambda b: (0, 0)),
]

_mm_relu = pl.pallas_call(
    _mm_relu_body,
    grid=(NB,),
    in_specs=_common_in_specs,
    out_specs=pl.BlockSpec((2, BN, HALF), lambda b: (0, b, 0)),
    out_shape=jax.ShapeDtypeStruct((2, N, HALF), jnp.float32),
)

_mm_pool_head = pl.pallas_call(
    _mm_pool_head_body,
    grid=(NB,),
    in_specs=_common_in_specs + [
        pl.BlockSpec((BN, 1), lambda b: (b, 0)),
        pl.BlockSpec((H, HALF), lambda b: (0, 0)),
        pl.BlockSpec((1, HALF), lambda b: (0, 0)),
    ],
    out_specs=pl.BlockSpec((G, HALF), lambda b: (0, 0)),
    out_shape=jax.ShapeDtypeStruct((G, HALF), jnp.float32),
    scratch_shapes=[
        pltpu.VMEM((G, H), jnp.float32),
        pltpu.VMEM((G, HALF), jnp.float32),
    ],
)


def kernel(x, edge_index, batch, Wl0, bl0, Wr0, Wl1, bl1, Wr1, Wl2, bl2, Wr2,
           Wl3, bl3, Wr3, head_W, head_b):
    _sc_degree, _sc_agg = _build_sc_kernels()
    src = edge_index[0].astype(jnp.int32)
    dst = edge_index[1].astype(jnp.int32)
    src_pad = jnp.concatenate(
        [src.reshape(NS, EPT), jnp.zeros((NS, EPTP - EPT), jnp.int32)], axis=1)
    src2 = jnp.concatenate([src_pad, src_pad + N], axis=0)
    dst_flat = jnp.concatenate(
        [dst.reshape(NS, EPT), jnp.full((NS, EPTP - EPT), N, jnp.int32)],
        axis=1).reshape(NS * EPTP)
    dst_rs = dst.reshape(NS, NCH, CH)

    ones_rows = jnp.ones((CH, HALF), jnp.float32)
    zeros_half = jnp.zeros((RPT, HALF), jnp.float32)

    deg2 = _sc_degree(dst_rs, ones_rows, zeros_half).reshape(NC, N, HALF)

    hflat = jnp.concatenate([x[:, 0:HALF], x[:, HALF:H]], axis=0)

    batch2d = batch.astype(jnp.int32).reshape(N, 1)
    hW_pad = jnp.pad(head_W, ((0, 0), (0, HALF - 1)))
    hb_pad = jnp.pad(head_b.reshape(1, 1), ((0, 0), (0, HALF - 1)))

    layers = ((Wl0, bl0, Wr0), (Wl1, bl1, Wr1), (Wl2, bl2, Wr2))
    for Wl, bl, Wr in layers:
        agg = _sc_agg(hflat, src2, dst_flat, zeros_half)
        hflat = _mm_relu(agg, agg, deg2, deg2, hflat, hflat,
                         Wl, bl.reshape(1, H), Wr).reshape(NC * N, HALF)

    agg = _sc_agg(hflat, src2, dst_flat, zeros_half)
    out = _mm_pool_head(agg, agg, deg2, deg2, hflat, hflat,
                        Wl3, bl3.reshape(1, H), Wr3, batch2d, hW_pad, hb_pad)
    return out[:, 0:1]


# speedup vs baseline: 1.7926x; 1.1406x over previous
import functools

import jax
import jax.numpy as jnp
from jax import lax
from jax.experimental import pallas as pl
from jax.experimental.pallas import tpu as pltpu
from jax.experimental.pallas import tpu_sc as plsc

N = 10000
E = 160000
H = 256
G = 64
HALF = 128
NC = 2
NS = 16
EPT = E // NS
CH = 80
NCH = EPT // CH
CHP = 80
EPTP = EPT
NCHP = EPTP // CHP
NPAD = N + 16
RPT = 624
TAIL = N - NS * RPT
NBUF = 2
BN = 1000
NB = N // BN

def _sc_degree_body(dst_hbm, ones_hbm, zeros_hbm, deg_hbm, dst_v, ones_v, deg_sh):
    c = lax.axis_index("c")
    s = lax.axis_index("s")
    pltpu.sync_copy(zeros_hbm, deg_sh.at[pl.ds(s * RPT, RPT)])

    @pl.when(s == 0)
    def _():
        pltpu.sync_copy(
            zeros_hbm.at[pl.ds(0, TAIL)], deg_sh.at[pl.ds(NS * RPT, TAIL)]
        )

    pltpu.sync_copy(dst_hbm.at[s], dst_v)
    pltpu.sync_copy(ones_hbm, ones_v)
    plsc.subcore_barrier()

    @pl.loop(0, NCH // 2)
    def _(j):
        pltpu.sync_copy(ones_v, deg_sh.at[dst_v.at[2 * j + c]], add=True)

    @pl.when(c == 0)
    def _():
        pltpu.sync_copy(ones_v, deg_sh.at[dst_v.at[NCH - 1]], add=True)

    plsc.subcore_barrier()
    pltpu.sync_copy(
        deg_sh.at[pl.ds(s * RPT, RPT)],
        deg_hbm.at[pl.ds(c * N + s * RPT, RPT)],
    )

    @pl.when(s == 0)
    def _():
        pltpu.sync_copy(
            deg_sh.at[pl.ds(NS * RPT, TAIL)],
            deg_hbm.at[pl.ds(c * N + NS * RPT, TAIL)],
        )


def _sc_agg_body(hflat_hbm, src2_hbm, dst_hbm, zeros_hbm, agg_hbm,
                 src_v, dst_v, rows_0, rows_1, acc_sh, sem_0, sem_1):
    c = lax.axis_index("c")
    s = lax.axis_index("s")
    rows = (rows_0, rows_1)
    sems = (sem_0, sem_1)
    pltpu.sync_copy(zeros_hbm, acc_sh.at[pl.ds(s * RPT, RPT)])

    @pl.when(s == 0)
    def _():
        pltpu.sync_copy(
            zeros_hbm.at[pl.ds(0, TAIL)], acc_sh.at[pl.ds(NS * RPT, TAIL)]
        )

    pltpu.sync_copy(src2_hbm.at[c * NS + s], src_v)
    pltpu.sync_copy(dst_hbm.at[s], dst_v)
    plsc.subcore_barrier()

    for b in range(NBUF):
        pltpu.async_copy(
            hflat_hbm.at[src_v.at[pl.ds(b * CHP, CHP)]], rows[b], sems[b]
        )

    @pl.loop(0, NCHP // NBUF)
    def _(j):
        for b in range(NBUF):
            k = NBUF * j + b
            pltpu.make_async_copy(
                hflat_hbm.at[src_v.at[pl.ds(k * CHP, CHP)]], rows[b], sems[b]
            ).wait()
            pltpu.sync_copy(rows[b], acc_sh.at[dst_v.at[k]], add=True)

            @pl.when(k + NBUF < NCHP)
            def _(b=b, k=k):
                pltpu.async_copy(
                    hflat_hbm.at[src_v.at[pl.ds((k + NBUF) * CHP, CHP)]],
                    rows[b], sems[b],
                )

    pltpu.make_async_copy(
        hflat_hbm.at[src_v.at[pl.ds((NCHP - 1) * CHP, CHP)]], rows[0], sems[0]
    ).wait()
    pltpu.sync_copy(rows[0], acc_sh.at[dst_v.at[NCHP - 1]], add=True)

    plsc.subcore_barrier()
    pltpu.sync_copy(
        acc_sh.at[pl.ds(s * RPT, RPT)],
        agg_hbm.at[pl.ds(c * N + s * RPT, RPT)],
    )

    @pl.when(s == 0)
    def _():
        pltpu.sync_copy(
            acc_sh.at[pl.ds(NS * RPT, TAIL)],
            agg_hbm.at[pl.ds(c * N + NS * RPT, TAIL)],
        )


@functools.cache
def _build_sc_kernels():
    mesh = plsc.VectorSubcoreMesh(
        core_axis_name="c", subcore_axis_name="s", num_cores=NC, num_subcores=NS
    )
    sc_degree = pl.kernel(
        _sc_degree_body,
        out_type=jax.ShapeDtypeStruct((NC * N, HALF), jnp.float32),
        mesh=mesh,
        scratch_types=[
            pltpu.VMEM((NCH, CH), jnp.int32),
            pltpu.VMEM((CH, HALF), jnp.float32),
            pltpu.VMEM_SHARED((N, HALF), jnp.float32),
        ],
    )
    sc_agg = pl.kernel(
        _sc_agg_body,
        out_type=jax.ShapeDtypeStruct((NC * N, HALF), jnp.float32),
        mesh=mesh,
        scratch_types=(
            [pltpu.VMEM((EPTP,), jnp.int32)]
            + [pltpu.VMEM((NCHP, CHP), jnp.int32)]
            + [pltpu.VMEM((CHP, HALF), jnp.float32)] * NBUF
            + [pltpu.VMEM_SHARED((N, HALF), jnp.float32)]
            + [pltpu.SemaphoreType.DMA] * NBUF
        ),
    )
    return sc_degree, sc_agg


def _dense_block(a0, a1, dg0, dg1, h0, h1, Wl, bl, Wr):
    deg = jnp.maximum(dg0[0][:, 0:1] + dg1[0][:, 0:1], 1.0)
    inv = 1.0 / deg
    return (
        jnp.dot(a0[...] * inv, Wl[0:HALF, :], preferred_element_type=jnp.float32)
        + jnp.dot(a1[...] * inv, Wl[HALF:H, :], preferred_element_type=jnp.float32)
        + jnp.dot(h0[...], Wr[0:HALF, :], preferred_element_type=jnp.float32)
        + jnp.dot(h1[...], Wr[HALF:H, :], preferred_element_type=jnp.float32)
        + bl[...]
    )


def _mm_relu_body(a0, a1, dg0, dg1, h0, h1, Wl, bl, Wr, out):
    acc = jnp.maximum(_dense_block(a0, a1, dg0, dg1, h0, h1, Wl, bl, Wr), 0.0)
    out[0] = acc[:, 0:HALF]
    out[1] = acc[:, HALF:H]


def _mm_pool_head_body(a0, a1, dg0, dg1, h0, h1, Wl, bl, Wr, batch, hW, hb,
                       out, pooled_acc, cnt_acc):
    b = pl.program_id(0)
    acc = _dense_block(a0, a1, dg0, dg1, h0, h1, Wl, bl, Wr)
    onehot = (
        batch[...] == lax.broadcasted_iota(jnp.int32, (BN, G), 1)
    ).astype(jnp.float32)

    @pl.when(b == 0)
    def _():
        pooled_acc[...] = jnp.zeros_like(pooled_acc)
        cnt_acc[...] = jnp.zeros_like(cnt_acc)

    pooled_acc[...] += lax.dot_general(
        onehot, acc, (((0,), (0,)), ((), ())),
        preferred_element_type=jnp.float32,
    )
    cnt_acc[...] += lax.dot_general(
        onehot, jnp.ones((BN, HALF), jnp.float32), (((0,), (0,)), ((), ())),
        preferred_element_type=jnp.float32,
    )

    @pl.when(b == NB - 1)
    def _():
        cnt = jnp.maximum(cnt_acc[:, 0:1], 1.0)
        pooled = pooled_acc[...] / cnt
        out[...] = (
            jnp.dot(pooled, hW[...], preferred_element_type=jnp.float32) + hb[...]
        )


def _row_spec(off):
    return pl.BlockSpec((BN, HALF), lambda b, off=off: (b + off, 0))


_common_in_specs = [
    _row_spec(0),
    _row_spec(NB),
    pl.BlockSpec((1, BN, HALF), lambda b: (0, b, 0)),
    pl.BlockSpec((1, BN, HALF), lambda b: (1, b, 0)),
    _row_spec(0),
    _row_spec(NB),
    pl.BlockSpec((H, H), lambda b: (0, 0)),
    pl.BlockSpec((1, H), lambda b: (0, 0)),
    pl.BlockSpec((H, H), lambda b: (0, 0)),
]

_mm_relu = pl.pallas_call(
    _mm_relu_body,
    grid=(NB,),
    in_specs=_common_in_specs,
    out_specs=pl.BlockSpec((2, BN, HALF), lambda b: (0, b, 0)),
    out_shape=jax.ShapeDtypeStruct((2, N, HALF), jnp.float32),
)

_mm_pool_head = pl.pallas_call(
    _mm_pool_head_body,
    grid=(NB,),
    in_specs=_common_in_specs + [
        pl.BlockSpec((BN, 1), lambda b: (b, 0)),
        pl.BlockSpec((H, HALF), lambda b: (0, 0)),
        pl.BlockSpec((1, HALF), lambda b: (0, 0)),
    ],
    out_specs=pl.BlockSpec((G, HALF), lambda b: (0, 0)),
    out_shape=jax.ShapeDtypeStruct((G, HALF), jnp.float32),
    scratch_shapes=[
        pltpu.VMEM((G, H), jnp.float32),
        pltpu.VMEM((G, HALF), jnp.float32),
    ],
)


def kernel(x, edge_index, batch, Wl0, bl0, Wr0, Wl1, bl1, Wr1, Wl2, bl2, Wr2,
           Wl3, bl3, Wr3, head_W, head_b):
    _sc_degree, _sc_agg = _build_sc_kernels()
    src = edge_index[0].astype(jnp.int32)
    dst = edge_index[1].astype(jnp.int32)
    src_rs = src.reshape(NS, EPT)
    src2 = jnp.concatenate([src_rs, src_rs + N], axis=0)
    dst_agg = dst.reshape(NS, NCHP, CHP)
    dst_rs = dst.reshape(NS, NCH, CH)

    ones_rows = jnp.ones((CH, HALF), jnp.float32)
    zeros_half = jnp.zeros((RPT, HALF), jnp.float32)

    deg2 = _sc_degree(dst_rs, ones_rows, zeros_half).reshape(NC, N, HALF)

    hflat = jnp.concatenate([x[:, 0:HALF], x[:, HALF:H]], axis=0)

    batch2d = batch.astype(jnp.int32).reshape(N, 1)
    hW_pad = jnp.pad(head_W, ((0, 0), (0, HALF - 1)))
    hb_pad = jnp.pad(head_b.reshape(1, 1), ((0, 0), (0, HALF - 1)))

    layers = ((Wl0, bl0, Wr0), (Wl1, bl1, Wr1), (Wl2, bl2, Wr2))
    for Wl, bl, Wr in layers:
        agg = _sc_agg(hflat, src2, dst_agg, zeros_half)
        hflat = _mm_relu(agg, agg, deg2, deg2, hflat, hflat,
                         Wl, bl.reshape(1, H), Wr).reshape(NC * N, HALF)

    agg = _sc_agg(hflat, src2, dst_agg, zeros_half)
    out = _mm_pool_head(agg, agg, deg2, deg2, hflat, hflat,
                        Wl3, bl3.reshape(1, H), Wr3, batch2d, hW_pad, hb_pad)
    return out[:, 0:1]
